# baseline (speedup 1.0000x reference)
import functools

import jax
import jax.numpy as jnp
from jax import lax
from jax.experimental import pallas as pl
from jax.experimental.pallas import tpu as pltpu
from jax.experimental.pallas import tpu_sc as plsc

N_SEG = 10000
ACC_ROWS = 10240
D = 128
NC = 2
NS = 16
LANES = 16

CHUNK = 80
ZROWS = 128


def _sc_partials(data, ids):
    n_edges = data.shape[0]
    per_worker = n_edges // (NC * NS)
    n_chunks = per_worker // CHUNK
    stripe = ACC_ROWS // NS

    mesh = plsc.VectorSubcoreMesh(
        core_axis_name="c", subcore_axis_name="s",
        num_cores=NC, num_subcores=NS)

    @functools.partial(
        pl.kernel,
        out_type=jax.ShapeDtypeStruct((NC, ACC_ROWS, D), jnp.float32),
        mesh=mesh,
        scratch_types=[
            pltpu.VMEM_SHARED((ACC_ROWS, D), jnp.float32),
            pltpu.VMEM((CHUNK, D), jnp.float32),
            pltpu.VMEM((CHUNK, D), jnp.float32),
            pltpu.VMEM((CHUNK,), jnp.int32),
            pltpu.VMEM((CHUNK,), jnp.int32),
            pltpu.VMEM((ZROWS, D), jnp.float32),
            pltpu.SemaphoreType.DMA,
            pltpu.SemaphoreType.DMA,
            pltpu.SemaphoreType.DMA,
            pltpu.SemaphoreType.DMA,
            pltpu.SemaphoreType.DMA,
            pltpu.SemaphoreType.DMA,
        ],
    )
    def k(data_hbm, ids_hbm, part_hbm, acc,
          buf_a, buf_b, idx_a, idx_b, zbuf,
          sda, sdb, sia, sib, ssa, ssb):
        cid = lax.axis_index("c")
        sid = lax.axis_index("s")

        def zstore(t, carry):
            r = t // (D // LANES)
            j = t % (D // LANES)
            zbuf[r, pl.ds(j * LANES, LANES)] = jnp.zeros((LANES,), jnp.float32)
            return carry
        lax.fori_loop(0, ZROWS * (D // LANES), zstore, 0)
        for r in range(stripe // ZROWS):
            pltpu.sync_copy(
                zbuf, acc.at[pl.ds(sid * stripe + r * ZROWS, ZROWS), :])
        plsc.subcore_barrier()

        base = (cid * NS + sid) * per_worker

        def fetch(kk, buf, idx, sd, si):
            off = base + kk * CHUNK
            pltpu.async_copy(data_hbm.at[pl.ds(off, CHUNK), :], buf, sd)
            pltpu.async_copy(ids_hbm.at[pl.ds(off, CHUNK)], idx, si)

        def wait_fetch(buf, idx, sd, si):
            pltpu.make_async_copy(data_hbm.at[pl.ds(base, CHUNK), :], buf, sd).wait()
            pltpu.make_async_copy(ids_hbm.at[pl.ds(base, CHUNK)], idx, si).wait()

        def scatter_start(buf, idx, ss):
            pltpu.async_copy(buf, acc.at[idx], ss, add=True)

        def scatter_wait(buf, idx, ss):
            pltpu.make_async_copy(buf, acc.at[idx], ss).wait()

        fetch(0, buf_a, idx_a, sda, sia)
        fetch(1, buf_b, idx_b, sdb, sib)

        def pair_body(t, carry):
            kk = 2 * t
            wait_fetch(buf_a, idx_a, sda, sia)
            scatter_start(buf_a, idx_a, ssa)
            wait_fetch(buf_b, idx_b, sdb, sib)
            scatter_start(buf_b, idx_b, ssb)

            scatter_wait(buf_a, idx_a, ssa)

            @pl.when(kk + 2 < n_chunks)
            def _():
                fetch(kk + 2, buf_a, idx_a, sda, sia)

            scatter_wait(buf_b, idx_b, ssb)

            @pl.when(kk + 3 < n_chunks)
            def _():
                fetch(kk + 3, buf_b, idx_b, sdb, sib)
            return carry
        lax.fori_loop(0, n_chunks // 2, pair_body, 0)

        wait_fetch(buf_a, idx_a, sda, sia)
        scatter_start(buf_a, idx_a, ssa)
        scatter_wait(buf_a, idx_a, ssa)

        plsc.subcore_barrier()
        pltpu.sync_copy(
            acc.at[pl.ds(sid * stripe, stripe), :],
            part_hbm.at[cid, pl.ds(sid * stripe, stripe), :])

    return k(data, ids)


def _tc_sum(partials):
    blk = N_SEG // 10

    def body(p_ref, o_ref):
        o_ref[...] = p_ref[0] + p_ref[1]

    return pl.pallas_call(
        body,
        out_shape=jax.ShapeDtypeStruct((N_SEG, D), jnp.float32),
        grid=(N_SEG // blk,),
        in_specs=[pl.BlockSpec((NC, blk, D), lambda i: (0, i, 0))],
        out_specs=pl.BlockSpec((blk, D), lambda i: (i, 0)),
    )(partials)


def kernel(data, segment_ids):
    ids = segment_ids.astype(jnp.int32)
    parts = _sc_partials(data, ids)
    return _tc_sum(parts)


# speedup vs baseline: 1.3172x; 1.3172x over previous
import functools

import jax
import jax.numpy as jnp
from jax import lax
from jax.experimental import pallas as pl
from jax.experimental.pallas import tpu as pltpu
from jax.experimental.pallas import tpu_sc as plsc

N_SEG = 10000
ACC_ROWS = 10240
D = 128
NC = 2
NS = 16
LANES = 16

CHUNK = 128
TAIL = 10000 - (10000 // CHUNK) * CHUNK


def _sc_partials(data, ids):
    n_edges = data.shape[0]
    per_worker = n_edges // (NC * NS)
    n_chunks = per_worker // CHUNK
    stripe = ACC_ROWS // NS

    mesh = plsc.VectorSubcoreMesh(
        core_axis_name="c", subcore_axis_name="s",
        num_cores=NC, num_subcores=NS)

    @functools.partial(
        pl.kernel,
        out_type=jax.ShapeDtypeStruct((NC, ACC_ROWS, D), jnp.float32),
        mesh=mesh,
        scratch_types=[
            pltpu.VMEM_SHARED((ACC_ROWS, D), jnp.float32),
            pltpu.VMEM((CHUNK, D), jnp.float32),
            pltpu.VMEM((CHUNK, D), jnp.float32),
            pltpu.VMEM((TAIL, D), jnp.float32),
            pltpu.VMEM((CHUNK,), jnp.int32),
            pltpu.VMEM((CHUNK,), jnp.int32),
            pltpu.VMEM((TAIL,), jnp.int32),
            pltpu.SemaphoreType.DMA,
            pltpu.SemaphoreType.DMA,
            pltpu.SemaphoreType.DMA,
            pltpu.SemaphoreType.DMA,
            pltpu.SemaphoreType.DMA,
            pltpu.SemaphoreType.DMA,
            pltpu.SemaphoreType.DMA,
        ],
    )
    def k(data_hbm, ids_hbm, part_hbm, acc,
          buf_a, buf_b, buf_t, idx_a, idx_b, idx_t,
          sda, sdb, sia, sib, ssa, ssb, sst):
        cid = lax.axis_index("c")
        sid = lax.axis_index("s")

        def zstore(t, carry):
            r = t // (D // LANES)
            j = t % (D // LANES)
            buf_a[r, pl.ds(j * LANES, LANES)] = jnp.zeros((LANES,), jnp.float32)
            return carry
        lax.fori_loop(0, CHUNK * (D // LANES), zstore, 0)
        for r in range(stripe // CHUNK):
            pltpu.sync_copy(
                buf_a, acc.at[pl.ds(sid * stripe + r * CHUNK, CHUNK), :])

        base = (cid * NS + sid) * per_worker

        def fetch(kk, buf, idx, sd, si):
            off = base + kk * CHUNK
            pltpu.async_copy(data_hbm.at[pl.ds(off, CHUNK), :], buf, sd)
            pltpu.async_copy(ids_hbm.at[pl.ds(off, CHUNK)], idx, si)

        def wait_fetch(buf, idx, sd, si):
            pltpu.make_async_copy(data_hbm.at[pl.ds(base, CHUNK), :], buf, sd).wait()
            pltpu.make_async_copy(ids_hbm.at[pl.ds(base, CHUNK)], idx, si).wait()

        def scatter(buf, idx, ss):
            pltpu.async_copy(buf, acc.at[idx], ss, add=True)
            pltpu.make_async_copy(buf, acc.at[idx], ss).wait()

        tail_off = base + n_chunks * CHUNK
        pltpu.async_copy(data_hbm.at[pl.ds(tail_off, TAIL), :], buf_t, sst)
        fetch(0, buf_a, idx_a, sda, sia)
        fetch(1, buf_b, idx_b, sdb, sib)
        plsc.subcore_barrier()

        def pair_body(t, carry):
            kk = 2 * t
            wait_fetch(buf_a, idx_a, sda, sia)
            scatter(buf_a, idx_a, ssa)

            @pl.when(kk + 2 < n_chunks)
            def _():
                fetch(kk + 2, buf_a, idx_a, sda, sia)

            wait_fetch(buf_b, idx_b, sdb, sib)
            scatter(buf_b, idx_b, ssb)

            @pl.when(kk + 3 < n_chunks)
            def _():
                fetch(kk + 3, buf_b, idx_b, sdb, sib)
            return carry
        lax.fori_loop(0, n_chunks // 2, pair_body, 0)

        pltpu.make_async_copy(data_hbm.at[pl.ds(tail_off, TAIL), :], buf_t, sst).wait()
        pltpu.async_copy(ids_hbm.at[pl.ds(tail_off, TAIL)], idx_t, sst)
        pltpu.make_async_copy(ids_hbm.at[pl.ds(tail_off, TAIL)], idx_t, sst).wait()
        scatter(buf_t, idx_t, sst)

        plsc.subcore_barrier()
        pltpu.sync_copy(
            acc.at[pl.ds(sid * stripe, stripe), :],
            part_hbm.at[cid, pl.ds(sid * stripe, stripe), :])

    return k(data, ids)


def _tc_sum(partials):
    blk = N_SEG // 10

    def body(p_ref, o_ref):
        o_ref[...] = p_ref[0] + p_ref[1]

    return pl.pallas_call(
        body,
        out_shape=jax.ShapeDtypeStruct((N_SEG, D), jnp.float32),
        grid=(N_SEG // blk,),
        in_specs=[pl.BlockSpec((NC, blk, D), lambda i: (0, i, 0))],
        out_specs=pl.BlockSpec((blk, D), lambda i: (i, 0)),
    )(partials)


def kernel(data, segment_ids):
    ids = segment_ids.astype(jnp.int32)
    parts = _sc_partials(data, ids)
    return _tc_sum(parts)


# speedup vs baseline: 1.3819x; 1.0491x over previous
import functools

import jax
import jax.numpy as jnp
from jax import lax
from jax.experimental import pallas as pl
from jax.experimental.pallas import tpu as pltpu
from jax.experimental.pallas import tpu_sc as plsc

N_SEG = 10000
ACC_ROWS = 10240
D = 128
NC = 2
NS = 16
LANES = 16

CHUNK = 128
TAIL = 10000 - (10000 // CHUNK) * CHUNK


def _sc_partials(data, ids):
    n_edges = data.shape[0]
    per_worker = n_edges // (NC * NS)
    n_chunks = per_worker // CHUNK
    stripe = ACC_ROWS // NS

    mesh = plsc.VectorSubcoreMesh(
        core_axis_name="c", subcore_axis_name="s",
        num_cores=NC, num_subcores=NS)

    @functools.partial(
        pl.kernel,
        out_type=jax.ShapeDtypeStruct((NC, ACC_ROWS, D), jnp.float32),
        mesh=mesh,
        scratch_types=[
            pltpu.VMEM_SHARED((ACC_ROWS, D), jnp.float32),
            pltpu.VMEM((CHUNK, D), jnp.float32),
            pltpu.VMEM((CHUNK, D), jnp.float32),
            pltpu.VMEM((TAIL, D), jnp.float32),
            pltpu.VMEM((CHUNK,), jnp.int32),
            pltpu.VMEM((CHUNK,), jnp.int32),
            pltpu.VMEM((TAIL,), jnp.int32),
            pltpu.SemaphoreType.DMA,
            pltpu.SemaphoreType.DMA,
            pltpu.SemaphoreType.DMA,
            pltpu.SemaphoreType.DMA,
            pltpu.SemaphoreType.DMA,
            pltpu.SemaphoreType.DMA,
            pltpu.SemaphoreType.DMA,
        ],
    )
    def k(data_hbm, ids_hbm, part_hbm, acc,
          buf_a, buf_b, buf_t, idx_a, idx_b, idx_t,
          sda, sdb, sia, sib, ssa, ssb, sst):
        cid = lax.axis_index("c")
        sid = lax.axis_index("s")

        zeros16 = jnp.zeros((LANES,), jnp.float32)

        def zstore(r, carry):
            for j in range(D // LANES):
                buf_a[r, pl.ds(j * LANES, LANES)] = zeros16
            return carry
        lax.fori_loop(0, CHUNK, zstore, 0)
        for r in range(stripe // CHUNK):
            pltpu.async_copy(
                buf_a, acc.at[pl.ds(sid * stripe + r * CHUNK, CHUNK), :], ssa)
        for r in range(stripe // CHUNK):
            pltpu.make_async_copy(
                buf_a, acc.at[pl.ds(sid * stripe, CHUNK), :], ssa).wait()

        base = (cid * NS + sid) * per_worker

        def fetch(kk, buf, idx, sd, si):
            off = base + kk * CHUNK
            pltpu.async_copy(data_hbm.at[pl.ds(off, CHUNK), :], buf, sd)
            pltpu.async_copy(ids_hbm.at[pl.ds(off, CHUNK)], idx, si)

        def wait_fetch(buf, idx, sd, si):
            pltpu.make_async_copy(data_hbm.at[pl.ds(base, CHUNK), :], buf, sd).wait()
            pltpu.make_async_copy(ids_hbm.at[pl.ds(base, CHUNK)], idx, si).wait()

        def scatter(buf, idx, ss):
            pltpu.async_copy(buf, acc.at[idx], ss, add=True)
            pltpu.make_async_copy(buf, acc.at[idx], ss).wait()

        tail_off = base + n_chunks * CHUNK
        pltpu.async_copy(data_hbm.at[pl.ds(tail_off, TAIL), :], buf_t, sst)
        fetch(0, buf_a, idx_a, sda, sia)
        fetch(1, buf_b, idx_b, sdb, sib)
        plsc.subcore_barrier()

        def pair_body(t, carry):
            kk = 2 * t
            wait_fetch(buf_a, idx_a, sda, sia)
            scatter(buf_a, idx_a, ssa)

            @pl.when(kk + 2 < n_chunks)
            def _():
                fetch(kk + 2, buf_a, idx_a, sda, sia)

            wait_fetch(buf_b, idx_b, sdb, sib)
            scatter(buf_b, idx_b, ssb)

            @pl.when(kk + 3 < n_chunks)
            def _():
                fetch(kk + 3, buf_b, idx_b, sdb, sib)
            return carry
        lax.fori_loop(0, n_chunks // 2, pair_body, 0)

        pltpu.make_async_copy(data_hbm.at[pl.ds(tail_off, TAIL), :], buf_t, sst).wait()
        pltpu.async_copy(ids_hbm.at[pl.ds(tail_off, TAIL)], idx_t, sst)
        pltpu.make_async_copy(ids_hbm.at[pl.ds(tail_off, TAIL)], idx_t, sst).wait()
        scatter(buf_t, idx_t, sst)

        plsc.subcore_barrier()
        pltpu.sync_copy(
            acc.at[pl.ds(sid * stripe, stripe), :],
            part_hbm.at[cid, pl.ds(sid * stripe, stripe), :])

    return k(data, ids)


def _tc_sum(partials):
    def body(p_ref, o_ref):
        o_ref[...] = p_ref[0, :N_SEG] + p_ref[1, :N_SEG]

    return pl.pallas_call(
        body,
        out_shape=jax.ShapeDtypeStruct((N_SEG, D), jnp.float32),
    )(partials)


def kernel(data, segment_ids):
    ids = segment_ids.astype(jnp.int32)
    parts = _sc_partials(data, ids)
    return _tc_sum(parts)


# speedup vs baseline: 1.3867x; 1.0035x over previous
import functools

import jax
import jax.numpy as jnp
from jax import lax
from jax.experimental import pallas as pl
from jax.experimental.pallas import tpu as pltpu
from jax.experimental.pallas import tpu_sc as plsc

N_SEG = 10000
ACC_ROWS = 10240
D = 128
NC = 2
NS = 16
LANES = 16

CHUNK = 128
TAIL = 10000 - (10000 // CHUNK) * CHUNK


def _sc_partials(data, ids):
    n_edges = data.shape[0]
    per_worker = n_edges // (NC * NS)
    n_chunks = per_worker // CHUNK
    stripe = ACC_ROWS // NS

    mesh = plsc.VectorSubcoreMesh(
        core_axis_name="c", subcore_axis_name="s",
        num_cores=NC, num_subcores=NS)

    @functools.partial(
        pl.kernel,
        out_type=jax.ShapeDtypeStruct((NC, ACC_ROWS, D), jnp.float32),
        mesh=mesh,
        scratch_types=[
            pltpu.VMEM_SHARED((ACC_ROWS, D), jnp.float32),
            pltpu.VMEM((CHUNK, D), jnp.float32),
            pltpu.VMEM((CHUNK, D), jnp.float32),
            pltpu.VMEM((TAIL, D), jnp.float32),
            pltpu.VMEM((CHUNK,), jnp.int32),
            pltpu.VMEM((CHUNK,), jnp.int32),
            pltpu.VMEM((TAIL,), jnp.int32),
            pltpu.SemaphoreType.DMA,
            pltpu.SemaphoreType.DMA,
            pltpu.SemaphoreType.DMA,
            pltpu.SemaphoreType.DMA,
            pltpu.SemaphoreType.DMA,
            pltpu.SemaphoreType.DMA,
            pltpu.SemaphoreType.DMA,
        ],
    )
    def k(data_hbm, ids_hbm, part_hbm, acc,
          buf_a, buf_b, buf_t, idx_a, idx_b, idx_t,
          sda, sdb, sia, sib, ssa, ssb, sst):
        cid = lax.axis_index("c")
        sid = lax.axis_index("s")
        base = (cid * NS + sid) * per_worker
        tail_off = base + n_chunks * CHUNK

        def fetch(kk, buf, idx, sd, si):
            off = base + kk * CHUNK
            pltpu.async_copy(data_hbm.at[pl.ds(off, CHUNK), :], buf, sd)
            pltpu.async_copy(ids_hbm.at[pl.ds(off, CHUNK)], idx, si)

        pltpu.async_copy(data_hbm.at[pl.ds(tail_off, TAIL), :], buf_t, sst)
        fetch(1, buf_b, idx_b, sdb, sib)

        zeros16 = jnp.zeros((LANES,), jnp.float32)

        def zstore(r, carry):
            for j in range(D // LANES):
                buf_a[r, pl.ds(j * LANES, LANES)] = zeros16
            return carry
        lax.fori_loop(0, CHUNK, zstore, 0)
        for r in range(stripe // CHUNK):
            pltpu.async_copy(
                buf_a, acc.at[pl.ds(sid * stripe + r * CHUNK, CHUNK), :], ssa)
        for r in range(stripe // CHUNK):
            pltpu.make_async_copy(
                buf_a, acc.at[pl.ds(sid * stripe, CHUNK), :], ssa).wait()

        def wait_fetch(buf, idx, sd, si):
            pltpu.make_async_copy(data_hbm.at[pl.ds(base, CHUNK), :], buf, sd).wait()
            pltpu.make_async_copy(ids_hbm.at[pl.ds(base, CHUNK)], idx, si).wait()

        def scatter(buf, idx, ss):
            pltpu.async_copy(buf, acc.at[idx], ss, add=True)
            pltpu.make_async_copy(buf, acc.at[idx], ss).wait()

        fetch(0, buf_a, idx_a, sda, sia)
        plsc.subcore_barrier()

        def pair_body(t, carry):
            kk = 2 * t
            wait_fetch(buf_a, idx_a, sda, sia)
            scatter(buf_a, idx_a, ssa)

            @pl.when(kk + 2 < n_chunks)
            def _():
                fetch(kk + 2, buf_a, idx_a, sda, sia)

            wait_fetch(buf_b, idx_b, sdb, sib)
            scatter(buf_b, idx_b, ssb)

            @pl.when(kk + 3 < n_chunks)
            def _():
                fetch(kk + 3, buf_b, idx_b, sdb, sib)
            return carry
        lax.fori_loop(0, n_chunks // 2, pair_body, 0)

        pltpu.make_async_copy(data_hbm.at[pl.ds(tail_off, TAIL), :], buf_t, sst).wait()
        pltpu.async_copy(ids_hbm.at[pl.ds(tail_off, TAIL)], idx_t, sst)
        pltpu.make_async_copy(ids_hbm.at[pl.ds(tail_off, TAIL)], idx_t, sst).wait()
        scatter(buf_t, idx_t, sst)

        plsc.subcore_barrier()
        pltpu.sync_copy(
            acc.at[pl.ds(sid * stripe, stripe), :],
            part_hbm.at[cid, pl.ds(sid * stripe, stripe), :])

    return k(data, ids)


def _tc_sum(partials):
    def body(p_ref, o_ref):
        o_ref[...] = p_ref[0, :N_SEG] + p_ref[1, :N_SEG]

    return pl.pallas_call(
        body,
        out_shape=jax.ShapeDtypeStruct((N_SEG, D), jnp.float32),
    )(partials)


def kernel(data, segment_ids):
    ids = segment_ids.astype(jnp.int32)
    parts = _sc_partials(data, ids)
    return _tc_sum(parts)


# speedup vs baseline: 1.3997x; 1.0094x over previous
import functools

import jax
import jax.numpy as jnp
from jax import lax
from jax.experimental import pallas as pl
from jax.experimental.pallas import tpu as pltpu
from jax.experimental.pallas import tpu_sc as plsc

N_SEG = 10000
ACC_ROWS = 10240
D = 128
NC = 2
NS = 16
LANES = 16

CHUNK = 128
TAIL = 10000 - (10000 // CHUNK) * CHUNK


def _sc_partials(data, ids):
    n_edges = data.shape[0]
    per_worker = n_edges // (NC * NS)
    n_chunks = per_worker // CHUNK
    stripe = ACC_ROWS // NS

    mesh = plsc.VectorSubcoreMesh(
        core_axis_name="c", subcore_axis_name="s",
        num_cores=NC, num_subcores=NS)

    @functools.partial(
        pl.kernel,
        out_type=jax.ShapeDtypeStruct((NC, ACC_ROWS, D), jnp.float32),
        mesh=mesh,
        scratch_types=[
            pltpu.VMEM_SHARED((ACC_ROWS, D), jnp.float32),
            pltpu.VMEM((CHUNK, D), jnp.float32),
            pltpu.VMEM((CHUNK, D), jnp.float32),
            pltpu.VMEM((TAIL, D), jnp.float32),
            pltpu.VMEM((CHUNK,), jnp.int32),
            pltpu.VMEM((CHUNK,), jnp.int32),
            pltpu.VMEM((TAIL,), jnp.int32),
            pltpu.VMEM((32, D), jnp.float32),
            pltpu.SemaphoreType.DMA,
            pltpu.SemaphoreType.DMA,
            pltpu.SemaphoreType.DMA,
            pltpu.SemaphoreType.DMA,
            pltpu.SemaphoreType.DMA,
            pltpu.SemaphoreType.DMA,
            pltpu.SemaphoreType.DMA,
        ],
    )
    def k(data_hbm, ids_hbm, part_hbm, acc,
          buf_a, buf_b, buf_t, idx_a, idx_b, idx_t, zbuf,
          sda, sdb, sia, sib, ssa, ssb, sst):
        cid = lax.axis_index("c")
        sid = lax.axis_index("s")
        base = (cid * NS + sid) * per_worker
        tail_off = base + n_chunks * CHUNK

        def fetch(kk, buf, idx, sd, si):
            off = base + kk * CHUNK
            pltpu.async_copy(data_hbm.at[pl.ds(off, CHUNK), :], buf, sd)
            pltpu.async_copy(ids_hbm.at[pl.ds(off, CHUNK)], idx, si)

        pltpu.async_copy(data_hbm.at[pl.ds(tail_off, TAIL), :], buf_t, sst)
        fetch(0, buf_a, idx_a, sda, sia)
        fetch(1, buf_b, idx_b, sdb, sib)

        zeros16 = jnp.zeros((LANES,), jnp.float32)

        def zstore(r, carry):
            for j in range(D // LANES):
                zbuf[r, pl.ds(j * LANES, LANES)] = zeros16
            return carry
        lax.fori_loop(0, 32, zstore, 0)
        for r in range(stripe // 32):
            pltpu.async_copy(
                zbuf, acc.at[pl.ds(sid * stripe + r * 32, 32), :], ssa)
        for r in range(stripe // 32):
            pltpu.make_async_copy(
                zbuf, acc.at[pl.ds(sid * stripe, 32), :], ssa).wait()

        def wait_fetch(buf, idx, sd, si):
            pltpu.make_async_copy(data_hbm.at[pl.ds(base, CHUNK), :], buf, sd).wait()
            pltpu.make_async_copy(ids_hbm.at[pl.ds(base, CHUNK)], idx, si).wait()

        def scatter(buf, idx, ss):
            pltpu.async_copy(buf, acc.at[idx], ss, add=True)
            pltpu.make_async_copy(buf, acc.at[idx], ss).wait()

        plsc.subcore_barrier()

        def pair_body(t, carry):
            kk = 2 * t
            wait_fetch(buf_a, idx_a, sda, sia)
            scatter(buf_a, idx_a, ssa)

            @pl.when(kk + 2 < n_chunks)
            def _():
                fetch(kk + 2, buf_a, idx_a, sda, sia)

            wait_fetch(buf_b, idx_b, sdb, sib)
            scatter(buf_b, idx_b, ssb)

            @pl.when(kk + 3 < n_chunks)
            def _():
                fetch(kk + 3, buf_b, idx_b, sdb, sib)
            return carry
        lax.fori_loop(0, n_chunks // 2, pair_body, 0)

        pltpu.make_async_copy(data_hbm.at[pl.ds(tail_off, TAIL), :], buf_t, sst).wait()
        pltpu.async_copy(ids_hbm.at[pl.ds(tail_off, TAIL)], idx_t, sst)
        pltpu.make_async_copy(ids_hbm.at[pl.ds(tail_off, TAIL)], idx_t, sst).wait()
        scatter(buf_t, idx_t, sst)

        plsc.subcore_barrier()
        pltpu.sync_copy(
            acc.at[pl.ds(sid * stripe, stripe), :],
            part_hbm.at[cid, pl.ds(sid * stripe, stripe), :])

    return k(data, ids)


def _tc_sum(partials):
    def body(p_ref, o_ref):
        o_ref[...] = p_ref[0, :N_SEG] + p_ref[1, :N_SEG]

    return pl.pallas_call(
        body,
        out_shape=jax.ShapeDtypeStruct((N_SEG, D), jnp.float32),
    )(partials)


def kernel(data, segment_ids):
    ids = segment_ids.astype(jnp.int32)
    parts = _sc_partials(data, ids)
    return _tc_sum(parts)
